# baseline (speedup 1.0000x reference)
import jax
import jax.numpy as jnp
from jax import lax
from jax.experimental import pallas as pl
from jax.experimental.pallas import tpu as pltpu
from jax.experimental.pallas import tpu_sc as plsc

VOCAB1 = 1000001
HIDDEN = 64
BATCH = 4096
SEQ = 200
NTOK = BATCH * SEQ
NC, NS = 2, 16
NW = NC * NS
EPW = BATCH // NW
TPW = EPW * SEQ

CH = 2048
QC = CH // 4
NCH = -(-VOCAB1 // CH)
VPAD = NCH * CH


def _tc_transform_body(tt_ref, w4_ref, b_ref, out_ref):
    x = tt_ref[...].astype(jnp.bfloat16)
    x4 = jnp.concatenate([x[:, i * QC : (i + 1) * QC] for i in range(4)],
                         axis=0)
    y4 = lax.dot_general(
        x4,
        w4_ref[...],
        dimension_numbers=(((0,), (1,)), ((), ())),
        preferred_element_type=jnp.float32,
    )
    y4 = y4 + b_ref[...]
    lo = y4[:, :128]
    hi = y4[:, 128:]
    lo16 = lax.bitcast_convert_type(lo.astype(jnp.bfloat16), jnp.uint16)
    hi16 = lax.bitcast_convert_type(hi.astype(jnp.bfloat16), jnp.uint16)
    w32 = lo16.astype(jnp.uint32) | (hi16.astype(jnp.uint32) << 16)
    out_ref[...] = lax.bitcast_convert_type(w32, jnp.float32)


def _tc_transform(tableT, W4, b2):
    return pl.pallas_call(
        _tc_transform_body,
        grid=(NCH,),
        in_specs=[
            pl.BlockSpec((HIDDEN, CH), lambda i: (0, i)),
            pl.BlockSpec((4 * HIDDEN, 4 * HIDDEN), lambda i: (0, 0)),
            pl.BlockSpec((1, 4 * HIDDEN), lambda i: (0, 0)),
        ],
        out_specs=pl.BlockSpec((QC, 2 * HIDDEN), lambda i: (i, 0)),
        out_shape=jax.ShapeDtypeStruct((VPAD // 4, 2 * HIDDEN), jnp.float32),
    )(tableT, W4, b2)


_sc_mesh = plsc.VectorSubcoreMesh(core_axis_name="c", subcore_axis_name="s")
_G0 = 128
_G1 = SEQ - _G0


def _fire(t2_hbm, idx_v, rows_v, base, slot, sem):
    c0 = pltpu.async_copy(
        t2_hbm.at[idx_v.at[pl.ds(base, _G0)]],
        rows_v.at[slot, pl.ds(0, _G0)],
        sem,
    )
    c1 = pltpu.async_copy(
        t2_hbm.at[idx_v.at[pl.ds(base + _G0, _G1)]],
        rows_v.at[slot, pl.ds(_G0, _G1)],
        sem,
    )
    return c0, c1


def _wait_slot(t2_hbm, idx_v, rows_v, slot, sem):
    pltpu.make_async_copy(
        t2_hbm.at[idx_v.at[pl.ds(0, _G0)]],
        rows_v.at[slot, pl.ds(0, _G0)],
        sem,
    ).wait()
    pltpu.make_async_copy(
        t2_hbm.at[idx_v.at[pl.ds(0, _G1)]],
        rows_v.at[slot, pl.ds(_G0, _G1)],
        sem,
    ).wait()


def _row(rows_v, slot, r, q):
    return plsc.bitcast(rows_v[slot, r, pl.ds(16 * q, 16)], jnp.bfloat16)


def _pool_one(rows_v, slot, outbuf_v, e):
    def body(i, acc):
        r = 8 + i * 8
        for u in range(8):
            acc = tuple(
                jnp.maximum(acc[q], _row(rows_v, slot, r + u, q))
                for q in range(2)
            )
        return acc

    acc0 = tuple(_row(rows_v, slot, 0, q) for q in range(2))
    for u in range(1, 8):
        acc0 = tuple(
            jnp.maximum(acc0[q], _row(rows_v, slot, u, q)) for q in range(2)
        )
    acc = lax.fori_loop(0, (SEQ - 8) // 8, body, acc0)
    a0, b0 = plsc.unpack(acc[0], format=plsc.PackFormat.INTERLEAVED)
    a1, b1 = plsc.unpack(acc[1], format=plsc.PackFormat.INTERLEAVED)
    outbuf_v[e, pl.ds(0, 16)] = a0
    outbuf_v[e, pl.ds(16, 16)] = a1
    outbuf_v[e, pl.ds(32, 16)] = b0
    outbuf_v[e, pl.ds(48, 16)] = b1


def _sc_pool_body(tok_hbm, t2_hbm, out_hbm, idx_v, rows_v, outbuf_v,
                  sem_a, sem_b):
    wid = lax.axis_index("s") * NC + lax.axis_index("c")
    tb = wid * TPW
    pltpu.sync_copy(tok_hbm.at[pl.ds(tb, TPW)], idx_v)

    def remap(i, carry):
        vv = idx_v[pl.ds(i * 16, 16)]
        r = vv & 2047
        rr = r & 511
        q = r >> 9
        idx_v[pl.ds(i * 16, 16)] = (vv - r) + (rr << 2) + q
        return carry

    lax.fori_loop(0, TPW // 16, remap, 0)

    _fire(t2_hbm, idx_v, rows_v, 0, 0, sem_a)

    def pair(ep, carry):
        e_a = 2 * ep
        cb = _fire(t2_hbm, idx_v, rows_v, (e_a + 1) * SEQ, 1, sem_b)
        _wait_slot(t2_hbm, idx_v, rows_v, 0, sem_a)
        _pool_one(rows_v, 0, outbuf_v, e_a)

        @pl.when(ep + 1 < EPW // 2)
        def _():
            _fire(t2_hbm, idx_v, rows_v, (e_a + 2) * SEQ, 0, sem_a)

        for cp in cb:
            cp.wait()
        _pool_one(rows_v, 1, outbuf_v, e_a + 1)
        return carry

    lax.fori_loop(0, EPW // 2, pair, 0)
    pltpu.sync_copy(outbuf_v, out_hbm.at[pl.ds(wid * EPW, EPW)])


_sc_pool = pl.kernel(
    _sc_pool_body,
    out_type=jax.ShapeDtypeStruct((BATCH, HIDDEN), jnp.float32),
    mesh=_sc_mesh,
    scratch_types=[
        pltpu.VMEM((TPW,), jnp.int32),
        pltpu.VMEM((2, SEQ, HIDDEN // 2), jnp.float32),
        pltpu.VMEM((EPW, HIDDEN), jnp.float32),
        pltpu.SemaphoreType.DMA,
        pltpu.SemaphoreType.DMA,
    ],
    compiler_params=pltpu.CompilerParams(
        use_tc_tiling_on_sc=False, needs_layout_passes=False
    ),
)


def kernel(token_ids, table, W, b):
    tok = token_ids.astype(jnp.int32).reshape(NTOK)
    wt = W.astype(jnp.bfloat16)
    zb = jnp.zeros((HIDDEN, HIDDEN), dtype=jnp.bfloat16)
    W4 = jnp.block([
        [wt, zb, zb, zb],
        [zb, wt, zb, zb],
        [zb, zb, wt, zb],
        [zb, zb, zb, wt],
    ])
    W4p = jnp.concatenate(
        [W4[q * HIDDEN : q * HIDDEN + 32, :] for q in range(4)]
        + [W4[q * HIDDEN + 32 : (q + 1) * HIDDEN, :] for q in range(4)],
        axis=0,
    )
    b4 = jnp.concatenate([b[:32]] * 4 + [b[32:]] * 4).reshape(1, 4 * HIDDEN)
    t2p = _tc_transform(table.T, W4p, b4)
    t2 = t2p.reshape(VPAD, HIDDEN // 2)
    return _sc_pool(tok, t2)


# speedup vs baseline: 1.3803x; 1.3803x over previous
import jax
import jax.numpy as jnp
from jax import lax
from jax.experimental import pallas as pl
from jax.experimental.pallas import tpu as pltpu
from jax.experimental.pallas import tpu_sc as plsc

VOCAB1 = 1000001
HIDDEN = 64
BATCH = 4096
SEQ = 200
NTOK = BATCH * SEQ
NC, NS = 2, 16
NW = NC * NS
EPW = BATCH // NW
TPW = EPW * SEQ

CH = 4096
QC = CH // 4
NCH = -(-VOCAB1 // CH)
VPAD = NCH * CH


def _tc_transform_body(tt_ref, w4_ref, b_ref, out_ref):
    x = tt_ref[...].astype(jnp.bfloat16)
    x4 = jnp.concatenate([x[:, i * QC : (i + 1) * QC] for i in range(4)],
                         axis=0)
    y4 = lax.dot_general(
        x4,
        w4_ref[...],
        dimension_numbers=(((0,), (1,)), ((), ())),
        preferred_element_type=jnp.float32,
    )
    y4 = y4 + b_ref[...]
    lo = y4[:, :128]
    hi = y4[:, 128:]
    lo16 = lax.bitcast_convert_type(lo.astype(jnp.bfloat16), jnp.uint16)
    hi16 = lax.bitcast_convert_type(hi.astype(jnp.bfloat16), jnp.uint16)
    w32 = lo16.astype(jnp.uint32) | (hi16.astype(jnp.uint32) << 16)
    out_ref[...] = lax.bitcast_convert_type(w32, jnp.float32)


def _tc_transform(tableT, W4, b2):
    return pl.pallas_call(
        _tc_transform_body,
        grid=(NCH,),
        in_specs=[
            pl.BlockSpec((HIDDEN, CH), lambda i: (0, i)),
            pl.BlockSpec((4 * HIDDEN, 4 * HIDDEN), lambda i: (0, 0)),
            pl.BlockSpec((1, 4 * HIDDEN), lambda i: (0, 0)),
        ],
        out_specs=pl.BlockSpec((QC, 2 * HIDDEN), lambda i: (i, 0)),
        out_shape=jax.ShapeDtypeStruct((VPAD // 4, 2 * HIDDEN), jnp.float32),
    )(tableT, W4, b2)


_sc_mesh = plsc.VectorSubcoreMesh(core_axis_name="c", subcore_axis_name="s")
_G0 = 128
_G1 = SEQ - _G0


def _fire(t2_hbm, idx_v, rows_v, base, slot, sem):
    c0 = pltpu.async_copy(
        t2_hbm.at[idx_v.at[pl.ds(base, _G0)]],
        rows_v.at[slot, pl.ds(0, _G0)],
        sem,
    )
    c1 = pltpu.async_copy(
        t2_hbm.at[idx_v.at[pl.ds(base + _G0, _G1)]],
        rows_v.at[slot, pl.ds(_G0, _G1)],
        sem,
    )
    return c0, c1


def _wait_slot(t2_hbm, idx_v, rows_v, slot, sem):
    pltpu.make_async_copy(
        t2_hbm.at[idx_v.at[pl.ds(0, _G0)]],
        rows_v.at[slot, pl.ds(0, _G0)],
        sem,
    ).wait()
    pltpu.make_async_copy(
        t2_hbm.at[idx_v.at[pl.ds(0, _G1)]],
        rows_v.at[slot, pl.ds(_G0, _G1)],
        sem,
    ).wait()


def _row(rows_v, slot, r, q):
    return plsc.bitcast(rows_v[slot, r, pl.ds(16 * q, 16)], jnp.bfloat16)


def _pool_one(rows_v, slot, outbuf_v, e):
    def body(i, acc):
        r = 8 + i * 8
        for u in range(8):
            acc = tuple(
                jnp.maximum(acc[q], _row(rows_v, slot, r + u, q))
                for q in range(2)
            )
        return acc

    acc0 = tuple(_row(rows_v, slot, 0, q) for q in range(2))
    for u in range(1, 8):
        acc0 = tuple(
            jnp.maximum(acc0[q], _row(rows_v, slot, u, q)) for q in range(2)
        )
    acc = lax.fori_loop(0, (SEQ - 8) // 8, body, acc0)
    a0, b0 = plsc.unpack(acc[0], format=plsc.PackFormat.INTERLEAVED)
    a1, b1 = plsc.unpack(acc[1], format=plsc.PackFormat.INTERLEAVED)
    outbuf_v[e, pl.ds(0, 16)] = a0
    outbuf_v[e, pl.ds(16, 16)] = a1
    outbuf_v[e, pl.ds(32, 16)] = b0
    outbuf_v[e, pl.ds(48, 16)] = b1


def _sc_pool_body(tok_hbm, t2_hbm, out_hbm, idx_v, rows_v, outbuf_v,
                  sem_a, sem_b):
    wid = lax.axis_index("s") * NC + lax.axis_index("c")
    tb = wid * TPW
    pltpu.sync_copy(tok_hbm.at[pl.ds(tb, TPW)], idx_v)

    qshift = QC.bit_length() - 1

    def remap(i, carry):
        vv = idx_v[pl.ds(i * 16, 16)]
        r = vv & (CH - 1)
        rr = r & (QC - 1)
        q = r >> qshift
        idx_v[pl.ds(i * 16, 16)] = (vv - r) + (rr << 2) + q
        return carry

    lax.fori_loop(0, TPW // 16, remap, 0)

    _fire(t2_hbm, idx_v, rows_v, 0, 0, sem_a)

    def pair(ep, carry):
        e_a = 2 * ep
        cb = _fire(t2_hbm, idx_v, rows_v, (e_a + 1) * SEQ, 1, sem_b)
        _wait_slot(t2_hbm, idx_v, rows_v, 0, sem_a)
        _pool_one(rows_v, 0, outbuf_v, e_a)

        @pl.when(ep + 1 < EPW // 2)
        def _():
            _fire(t2_hbm, idx_v, rows_v, (e_a + 2) * SEQ, 0, sem_a)

        for cp in cb:
            cp.wait()
        _pool_one(rows_v, 1, outbuf_v, e_a + 1)
        return carry

    lax.fori_loop(0, EPW // 2, pair, 0)
    pltpu.sync_copy(outbuf_v, out_hbm.at[pl.ds(wid * EPW, EPW)])


_sc_pool = pl.kernel(
    _sc_pool_body,
    out_type=jax.ShapeDtypeStruct((BATCH, HIDDEN), jnp.float32),
    mesh=_sc_mesh,
    scratch_types=[
        pltpu.VMEM((TPW,), jnp.int32),
        pltpu.VMEM((2, SEQ, HIDDEN // 2), jnp.float32),
        pltpu.VMEM((EPW, HIDDEN), jnp.float32),
        pltpu.SemaphoreType.DMA,
        pltpu.SemaphoreType.DMA,
    ],
    compiler_params=pltpu.CompilerParams(
        use_tc_tiling_on_sc=False, needs_layout_passes=False
    ),
)


def kernel(token_ids, table, W, b):
    tok = token_ids.astype(jnp.int32).reshape(NTOK)
    wt = W.astype(jnp.bfloat16)
    zb = jnp.zeros((HIDDEN, HIDDEN), dtype=jnp.bfloat16)
    W4 = jnp.block([
        [wt, zb, zb, zb],
        [zb, wt, zb, zb],
        [zb, zb, wt, zb],
        [zb, zb, zb, wt],
    ])
    W4p = jnp.concatenate(
        [W4[q * HIDDEN : q * HIDDEN + 32, :] for q in range(4)]
        + [W4[q * HIDDEN + 32 : (q + 1) * HIDDEN, :] for q in range(4)],
        axis=0,
    )
    b4 = jnp.concatenate([b[:32]] * 4 + [b[32:]] * 4).reshape(1, 4 * HIDDEN)
    t2p = _tc_transform(table.T, W4p, b4)
    t2 = t2p.reshape(VPAD, HIDDEN // 2)
    return _sc_pool(tok, t2)


# speedup vs baseline: 1.7203x; 1.2463x over previous
import jax
import jax.numpy as jnp
from jax import lax
from jax.experimental import pallas as pl
from jax.experimental.pallas import tpu as pltpu
from jax.experimental.pallas import tpu_sc as plsc

VOCAB1 = 1000001
HIDDEN = 64
BATCH = 4096
SEQ = 200
NTOK = BATCH * SEQ
NC, NS = 2, 16
NW = NC * NS
EPW = BATCH // NW
TPW = EPW * SEQ

CH = 8192
QC = CH // 4
NCH = -(-VOCAB1 // CH)
VPAD = NCH * CH


def _tc_transform_body(tt_ref, w4_ref, b_ref, out_ref):
    x = tt_ref[...].astype(jnp.bfloat16)
    x4 = jnp.concatenate([x[:, i * QC : (i + 1) * QC] for i in range(4)],
                         axis=0)
    y4 = lax.dot_general(
        x4,
        w4_ref[...],
        dimension_numbers=(((0,), (1,)), ((), ())),
        preferred_element_type=jnp.float32,
    )
    y4 = y4 + b_ref[...]
    lo = y4[:, :128]
    hi = y4[:, 128:]
    lo16 = lax.bitcast_convert_type(lo.astype(jnp.bfloat16), jnp.uint16)
    hi16 = lax.bitcast_convert_type(hi.astype(jnp.bfloat16), jnp.uint16)
    w32 = lo16.astype(jnp.uint32) | (hi16.astype(jnp.uint32) << 16)
    out_ref[...] = lax.bitcast_convert_type(w32, jnp.float32)


def _tc_transform(tableT, W4, b2):
    return pl.pallas_call(
        _tc_transform_body,
        grid=(NCH,),
        in_specs=[
            pl.BlockSpec((HIDDEN, CH), lambda i: (0, i)),
            pl.BlockSpec((4 * HIDDEN, 4 * HIDDEN), lambda i: (0, 0)),
            pl.BlockSpec((1, 4 * HIDDEN), lambda i: (0, 0)),
        ],
        out_specs=pl.BlockSpec((QC, 2 * HIDDEN), lambda i: (i, 0)),
        out_shape=jax.ShapeDtypeStruct((VPAD // 4, 2 * HIDDEN), jnp.float32),
    )(tableT, W4, b2)


_sc_mesh = plsc.VectorSubcoreMesh(core_axis_name="c", subcore_axis_name="s")
_G0 = 128
_G1 = SEQ - _G0


def _fire(t2_hbm, idx_v, rows_v, base, slot, sem):
    c0 = pltpu.async_copy(
        t2_hbm.at[idx_v.at[pl.ds(base, _G0)]],
        rows_v.at[slot, pl.ds(0, _G0)],
        sem,
    )
    c1 = pltpu.async_copy(
        t2_hbm.at[idx_v.at[pl.ds(base + _G0, _G1)]],
        rows_v.at[slot, pl.ds(_G0, _G1)],
        sem,
    )
    return c0, c1


def _wait_slot(t2_hbm, idx_v, rows_v, slot, sem):
    pltpu.make_async_copy(
        t2_hbm.at[idx_v.at[pl.ds(0, _G0)]],
        rows_v.at[slot, pl.ds(0, _G0)],
        sem,
    ).wait()
    pltpu.make_async_copy(
        t2_hbm.at[idx_v.at[pl.ds(0, _G1)]],
        rows_v.at[slot, pl.ds(_G0, _G1)],
        sem,
    ).wait()


def _row(rows_v, slot, r, q):
    return plsc.bitcast(rows_v[slot, r, pl.ds(16 * q, 16)], jnp.bfloat16)


def _pool_one(rows_v, slot, outbuf_v, e):
    def body(i, acc):
        r = 8 + i * 8
        for u in range(8):
            acc = tuple(
                jnp.maximum(acc[q], _row(rows_v, slot, r + u, q))
                for q in range(2)
            )
        return acc

    acc0 = tuple(_row(rows_v, slot, 0, q) for q in range(2))
    for u in range(1, 8):
        acc0 = tuple(
            jnp.maximum(acc0[q], _row(rows_v, slot, u, q)) for q in range(2)
        )
    acc = lax.fori_loop(0, (SEQ - 8) // 8, body, acc0)
    a0, b0 = plsc.unpack(acc[0], format=plsc.PackFormat.INTERLEAVED)
    a1, b1 = plsc.unpack(acc[1], format=plsc.PackFormat.INTERLEAVED)
    outbuf_v[e, pl.ds(0, 16)] = a0
    outbuf_v[e, pl.ds(16, 16)] = a1
    outbuf_v[e, pl.ds(32, 16)] = b0
    outbuf_v[e, pl.ds(48, 16)] = b1


def _sc_pool_body(tok_hbm, t2_hbm, out_hbm, idx_v, rows_v, outbuf_v,
                  sem_a, sem_b):
    wid = lax.axis_index("s") * NC + lax.axis_index("c")
    tb = wid * TPW
    pltpu.sync_copy(tok_hbm.at[pl.ds(tb, TPW)], idx_v)

    qshift = QC.bit_length() - 1

    def remap(i, carry):
        vv = idx_v[pl.ds(i * 16, 16)]
        r = vv & (CH - 1)
        rr = r & (QC - 1)
        q = r >> qshift
        idx_v[pl.ds(i * 16, 16)] = (vv - r) + (rr << 2) + q
        return carry

    lax.fori_loop(0, TPW // 16, remap, 0)

    _fire(t2_hbm, idx_v, rows_v, 0, 0, sem_a)

    def pair(ep, carry):
        e_a = 2 * ep
        cb = _fire(t2_hbm, idx_v, rows_v, (e_a + 1) * SEQ, 1, sem_b)
        _wait_slot(t2_hbm, idx_v, rows_v, 0, sem_a)
        _pool_one(rows_v, 0, outbuf_v, e_a)

        @pl.when(ep + 1 < EPW // 2)
        def _():
            _fire(t2_hbm, idx_v, rows_v, (e_a + 2) * SEQ, 0, sem_a)

        for cp in cb:
            cp.wait()
        _pool_one(rows_v, 1, outbuf_v, e_a + 1)
        return carry

    lax.fori_loop(0, EPW // 2, pair, 0)
    pltpu.sync_copy(outbuf_v, out_hbm.at[pl.ds(wid * EPW, EPW)])


_sc_pool = pl.kernel(
    _sc_pool_body,
    out_type=jax.ShapeDtypeStruct((BATCH, HIDDEN), jnp.float32),
    mesh=_sc_mesh,
    scratch_types=[
        pltpu.VMEM((TPW,), jnp.int32),
        pltpu.VMEM((2, SEQ, HIDDEN // 2), jnp.float32),
        pltpu.VMEM((EPW, HIDDEN), jnp.float32),
        pltpu.SemaphoreType.DMA,
        pltpu.SemaphoreType.DMA,
    ],
    compiler_params=pltpu.CompilerParams(
        use_tc_tiling_on_sc=False, needs_layout_passes=False
    ),
)


def kernel(token_ids, table, W, b):
    tok = token_ids.astype(jnp.int32).reshape(NTOK)
    wt = W.astype(jnp.bfloat16)
    zb = jnp.zeros((HIDDEN, HIDDEN), dtype=jnp.bfloat16)
    W4 = jnp.block([
        [wt, zb, zb, zb],
        [zb, wt, zb, zb],
        [zb, zb, wt, zb],
        [zb, zb, zb, wt],
    ])
    W4p = jnp.concatenate(
        [W4[q * HIDDEN : q * HIDDEN + 32, :] for q in range(4)]
        + [W4[q * HIDDEN + 32 : (q + 1) * HIDDEN, :] for q in range(4)],
        axis=0,
    )
    b4 = jnp.concatenate([b[:32]] * 4 + [b[32:]] * 4).reshape(1, 4 * HIDDEN)
    t2p = _tc_transform(table.T, W4p, b4)
    t2 = t2p.reshape(VPAD, HIDDEN // 2)
    return _sc_pool(tok, t2)
